# baseline (speedup 1.0000x reference)
import functools

import jax
import jax.numpy as jnp
from jax import lax
from jax.experimental import pallas as pl
from jax.experimental.pallas import tpu as pltpu
from jax.experimental.pallas import tpu_sc as plsc

_B = 16384
_F = 32
_NC = 2
_NS = 16
_NW = _NC * _NS
_BPW = _B // _NW
_CHUNK = 128
_NCH = _BPW // _CHUNK
_L = 16


def _body(user_hbm, movie_hbm, uf_hbm, mf_hbm, out_hbm,
          uidx, midx, urows, mrows, outv, sem):
    c = lax.axis_index("c")
    s = lax.axis_index("s")
    wid = s * _NC + c
    base = wid * _BPW

    for j in range(_NCH):
        pltpu.sync_copy(user_hbm.at[pl.ds(base + j * _CHUNK, _CHUNK)],
                        uidx.at[j])
        pltpu.sync_copy(movie_hbm.at[pl.ds(base + j * _CHUNK, _CHUNK)],
                        midx.at[j])

    copies = []
    for j in range(_NCH):
        copies.append(pltpu.async_copy(
            uf_hbm.at[uidx.at[j]], urows.at[pl.ds(j * _CHUNK, _CHUNK)], sem))
        copies.append(pltpu.async_copy(
            mf_hbm.at[midx.at[j]], mrows.at[pl.ds(j * _CHUNK, _CHUNK)], sem))
    for cp in copies:
        cp.wait()

    lane = lax.iota(jnp.int32, _L)
    cols = [jnp.full((_L,), f, jnp.int32) for f in range(_F)]

    def group(g, _):
        rows16 = g * _L + lane
        acc = None
        for f in range(_F):
            uv = plsc.load_gather(urows, [rows16, cols[f]])
            mv = plsc.load_gather(mrows, [rows16, cols[f]])
            p = uv * mv
            acc = p if acc is None else acc + p
        outv[pl.ds(g * _L, _L)] = acc
        return 0

    lax.fori_loop(0, _BPW // _L, group, 0)

    pltpu.sync_copy(outv, out_hbm.at[pl.ds(base, _BPW)])


_mesh = plsc.VectorSubcoreMesh(core_axis_name="c", subcore_axis_name="s")

_mf_call = functools.partial(
    pl.kernel,
    out_type=jax.ShapeDtypeStruct((_B,), jnp.float32),
    mesh=_mesh,
    scratch_types=[
        pltpu.VMEM((_NCH, _CHUNK), jnp.int32),
        pltpu.VMEM((_NCH, _CHUNK), jnp.int32),
        pltpu.VMEM((_BPW, _F), jnp.float32),
        pltpu.VMEM((_BPW, _F), jnp.float32),
        pltpu.VMEM((_BPW,), jnp.float32),
        pltpu.SemaphoreType.DMA,
    ],
    compiler_params=pltpu.CompilerParams(
        needs_layout_passes=False, use_tc_tiling_on_sc=False),
)(_body)


@jax.jit
def kernel(user, movie, user_factors, movie_factors):
    return _mf_call(user, movie, user_factors, movie_factors)
